# baseline (speedup 1.0000x reference)
import jax
import jax.numpy as jnp
from jax import lax
from jax.experimental import pallas as pl
from jax.experimental.pallas import tpu as pltpu
from jax.experimental.pallas import tpu_sc as plsc

C = 384
HW = 224 * 224
NC, NS, L = 2, 16, 16
NW = NC * NS
CPW = C // NW
NV = HW // L
STRIDE = 257
SUB = 1
NREG = L * SUB
HSCR = NREG * STRIDE + L

_GATHER_DNUMS = lax.GatherDimensionNumbers(
    offset_dims=(), collapsed_slice_dims=(0,), start_index_map=(0,))


def _lane_shuffle(v, idx):
    return lax.gather(v, idx[:, None], _GATHER_DNUMS, slice_sizes=(1,),
                      mode=lax.GatherScatterMode.PROMISE_IN_BOUNDS)


def _hist_body(inp, min_o, max_o, hist_o,
               buf0, buf1, histscr, histscr2, rowbuf, minbuf, maxbuf,
               sem0, sem1):
    wid = lax.axis_index("s") * NC + lax.axis_index("c")
    base_ch = wid * CPW
    lane = lax.iota(jnp.int32, L)
    ones = jnp.ones((L,), jnp.float32)
    zeros = jnp.zeros((L,), jnp.float32)
    region_f = [((lane + s * L) * STRIDE).astype(jnp.float32)
                for s in range(SUB)]

    def zbody(i):
        histscr[pl.ds(i * L, L)] = zeros
        histscr2[pl.ds(i * L, L)] = zeros
    plsc.parallel_loop(0, HSCR // L, unroll=4)(zbody)

    bufs = (buf0, buf1)
    sems = (sem0, sem1)
    cur_cp = pltpu.async_copy(inp.at[base_ch], buf0, sem0)

    for j in range(CPW):
        cur = bufs[j % 2]
        nxt_cp = None
        if j + 1 < CPW:
            nxt_cp = pltpu.async_copy(inp.at[base_ch + j + 1],
                                      bufs[(j + 1) % 2], sems[(j + 1) % 2])
        cur_cp.wait()

        v0 = cur[pl.ds(0, L)]

        def abody(i, carry):
            mn0, mn1, mn2, mn3, mx0, mx1, mx2, mx3 = carry
            a = cur[pl.ds(i * L, L)]
            b = cur[pl.ds((i + 1) * L, L)]
            c = cur[pl.ds((i + 2) * L, L)]
            d = cur[pl.ds((i + 3) * L, L)]
            return (jnp.minimum(mn0, a), jnp.minimum(mn1, b),
                    jnp.minimum(mn2, c), jnp.minimum(mn3, d),
                    jnp.maximum(mx0, a), jnp.maximum(mx1, b),
                    jnp.maximum(mx2, c), jnp.maximum(mx3, d))

        mn0, mn1, mn2, mn3, mx0, mx1, mx2, mx3 = plsc.parallel_loop(
            0, NV, step=4, unroll=2,
            carry=(v0, v0, v0, v0, v0, v0, v0, v0))(abody)
        mn = jnp.minimum(jnp.minimum(mn0, mn1), jnp.minimum(mn2, mn3))
        mx = jnp.maximum(jnp.maximum(mx0, mx1), jnp.maximum(mx2, mx3))
        for k in (1, 2, 4, 8):
            perm = lane ^ k
            mn = jnp.minimum(mn, _lane_shuffle(mn, perm))
            mx = jnp.maximum(mx, _lane_shuffle(mx, perm))
        rng = mx - mn
        denom = jnp.where(rng > 0.0, rng, jnp.full((L,), 1.0, jnp.float32))
        scale = jnp.full((L,), 255.99975585937500, jnp.float32) / denom
        mnsc = mn * scale
        biases = [rf - mnsc for rf in region_f]

        lm = lane == j
        minbuf[...] = jnp.where(lm, mn, minbuf[...])
        maxbuf[...] = jnp.where(lm, mx, maxbuf[...])

        def bbody(i):
            va = cur[pl.ds(i * L, L)]
            ta = va * scale + biases[0]
            plsc.addupdate_scatter(histscr, [ta.astype(jnp.int32)], ones)
            vb = cur[pl.ds((i + 1) * L, L)]
            tb = vb * scale + biases[0]
            plsc.addupdate_scatter(histscr2, [tb.astype(jnp.int32)], ones)
        plsc.parallel_loop(0, NV, step=2, unroll=4)(bbody)

        def cbody(g):
            acc = zeros
            for q in range(NREG):
                acc = acc + histscr[pl.ds(q * STRIDE + g * L, L)]
                histscr[pl.ds(q * STRIDE + g * L, L)] = zeros
                acc = acc + histscr2[pl.ds(q * STRIDE + g * L, L)]
                histscr2[pl.ds(q * STRIDE + g * L, L)] = zeros
            rowbuf[pl.ds(g * L, L)] = acc
        plsc.parallel_loop(0, 256 // L, unroll=2)(cbody)

        pltpu.sync_copy(rowbuf, hist_o.at[base_ch + j])
        cur_cp = nxt_cp

    pltpu.sync_copy(minbuf, min_o.at[wid])
    pltpu.sync_copy(maxbuf, max_o.at[wid])


_mesh = plsc.VectorSubcoreMesh(core_axis_name="c", subcore_axis_name="s",
                               num_cores=NC, num_subcores=NS)

_sc_hist = pl.kernel(
    _hist_body,
    out_type=(jax.ShapeDtypeStruct((NW, L), jnp.float32),
              jax.ShapeDtypeStruct((NW, L), jnp.float32),
              jax.ShapeDtypeStruct((C, 256), jnp.float32)),
    mesh=_mesh,
    scratch_types=(pltpu.VMEM((HW,), jnp.float32),
                   pltpu.VMEM((HW,), jnp.float32),
                   pltpu.VMEM((HSCR,), jnp.float32),
                   pltpu.VMEM((HSCR,), jnp.float32),
                   pltpu.VMEM((256,), jnp.float32),
                   pltpu.VMEM((L,), jnp.float32),
                   pltpu.VMEM((L,), jnp.float32),
                   pltpu.SemaphoreType.DMA,
                   pltpu.SemaphoreType.DMA),
    compiler_params=pltpu.CompilerParams(needs_layout_passes=False),
)


def kernel(input):
    x2 = input.reshape(C, HW)
    min_st, max_st, hist = _sc_hist(x2)
    target_min = min_st[:, :CPW].reshape(C)
    target_max = max_st[:, :CPW].reshape(C)
    return (input, hist, target_min, target_max)


# speedup vs baseline: 1.1222x; 1.1222x over previous
import jax
import jax.numpy as jnp
from jax import lax
from jax.experimental import pallas as pl
from jax.experimental.pallas import tpu as pltpu
from jax.experimental.pallas import tpu_sc as plsc

C = 384
HW = 224 * 224
NC, NS, L = 2, 16, 16
NW = NC * NS
CPW = C // NW
NV = HW // L
STRIDE = 257
SUB = 1
NREG = L * SUB
HSCR = NREG * STRIDE + L

_GATHER_DNUMS = lax.GatherDimensionNumbers(
    offset_dims=(), collapsed_slice_dims=(0,), start_index_map=(0,))


def _lane_shuffle(v, idx):
    return lax.gather(v, idx[:, None], _GATHER_DNUMS, slice_sizes=(1,),
                      mode=lax.GatherScatterMode.PROMISE_IN_BOUNDS)


def _hist_body(inp, min_o, max_o, hist_o,
               buf0, buf1, histscr, rowbuf, minbuf, maxbuf,
               sem0, sem1):
    wid = lax.axis_index("s") * NC + lax.axis_index("c")
    base_ch = wid * CPW
    lane = lax.iota(jnp.int32, L)
    ones = jnp.ones((L,), jnp.float32)
    zeros = jnp.zeros((L,), jnp.float32)
    region_f = [((lane + s * L) * STRIDE).astype(jnp.float32)
                for s in range(SUB)]

    izeros = jnp.zeros((L,), jnp.int32)

    def zbody(i):
        histscr[pl.ds(i * L, L)] = izeros
    plsc.parallel_loop(0, HSCR // L, unroll=4)(zbody)

    bufs = (buf0, buf1)
    sems = (sem0, sem1)
    cur_cp = pltpu.async_copy(inp.at[base_ch], buf0, sem0)

    for j in range(CPW):
        cur = bufs[j % 2]
        nxt_cp = None
        if j + 1 < CPW:
            nxt_cp = pltpu.async_copy(inp.at[base_ch + j + 1],
                                      bufs[(j + 1) % 2], sems[(j + 1) % 2])
        cur_cp.wait()

        v0 = cur[pl.ds(0, L)]

        def abody(i, carry):
            mn0, mn1, mn2, mn3, mx0, mx1, mx2, mx3 = carry
            a = cur[pl.ds(i * L, L)]
            b = cur[pl.ds((i + 1) * L, L)]
            c = cur[pl.ds((i + 2) * L, L)]
            d = cur[pl.ds((i + 3) * L, L)]
            return (jnp.minimum(mn0, a), jnp.minimum(mn1, b),
                    jnp.minimum(mn2, c), jnp.minimum(mn3, d),
                    jnp.maximum(mx0, a), jnp.maximum(mx1, b),
                    jnp.maximum(mx2, c), jnp.maximum(mx3, d))

        mn0, mn1, mn2, mn3, mx0, mx1, mx2, mx3 = plsc.parallel_loop(
            0, NV, step=4, unroll=2,
            carry=(v0, v0, v0, v0, v0, v0, v0, v0))(abody)
        mn = jnp.minimum(jnp.minimum(mn0, mn1), jnp.minimum(mn2, mn3))
        mx = jnp.maximum(jnp.maximum(mx0, mx1), jnp.maximum(mx2, mx3))
        for k in (1, 2, 4, 8):
            perm = lane ^ k
            mn = jnp.minimum(mn, _lane_shuffle(mn, perm))
            mx = jnp.maximum(mx, _lane_shuffle(mx, perm))
        rng = mx - mn
        denom = jnp.where(rng > 0.0, rng, jnp.full((L,), 1.0, jnp.float32))
        scale = jnp.full((L,), 255.99975585937500, jnp.float32) / denom
        mnsc = mn * scale
        biases = [rf - mnsc for rf in region_f]

        lm = lane == j
        minbuf[...] = jnp.where(lm, mn, minbuf[...])
        maxbuf[...] = jnp.where(lm, mx, maxbuf[...])

        iones = jnp.ones((L,), jnp.int32)

        def bbody(i):
            v = cur[pl.ds(i * L, L)]
            t = v * scale + biases[0]
            plsc.addupdate_scatter(histscr, [t.astype(jnp.int32)], iones)
        plsc.parallel_loop(0, NV, unroll=8)(bbody)

        def cbody(g):
            acc = jnp.zeros((L,), jnp.int32)
            for q in range(NREG):
                acc = acc + histscr[pl.ds(q * STRIDE + g * L, L)]
                histscr[pl.ds(q * STRIDE + g * L, L)] = izeros
            rowbuf[pl.ds(g * L, L)] = acc.astype(jnp.float32)
        plsc.parallel_loop(0, 256 // L, unroll=2)(cbody)

        pltpu.sync_copy(rowbuf, hist_o.at[base_ch + j])
        cur_cp = nxt_cp

    pltpu.sync_copy(minbuf, min_o.at[wid])
    pltpu.sync_copy(maxbuf, max_o.at[wid])


_mesh = plsc.VectorSubcoreMesh(core_axis_name="c", subcore_axis_name="s",
                               num_cores=NC, num_subcores=NS)

_sc_hist = pl.kernel(
    _hist_body,
    out_type=(jax.ShapeDtypeStruct((NW, L), jnp.float32),
              jax.ShapeDtypeStruct((NW, L), jnp.float32),
              jax.ShapeDtypeStruct((C, 256), jnp.float32)),
    mesh=_mesh,
    scratch_types=(pltpu.VMEM((HW,), jnp.float32),
                   pltpu.VMEM((HW,), jnp.float32),
                   pltpu.VMEM((HSCR,), jnp.int32),
                   pltpu.VMEM((256,), jnp.float32),
                   pltpu.VMEM((L,), jnp.float32),
                   pltpu.VMEM((L,), jnp.float32),
                   pltpu.SemaphoreType.DMA,
                   pltpu.SemaphoreType.DMA),
    compiler_params=pltpu.CompilerParams(needs_layout_passes=False),
)


def kernel(input):
    x2 = input.reshape(C, HW)
    min_st, max_st, hist = _sc_hist(x2)
    target_min = min_st[:, :CPW].reshape(C)
    target_max = max_st[:, :CPW].reshape(C)
    return (input, hist, target_min, target_max)


# speedup vs baseline: 1.1297x; 1.0067x over previous
import jax
import jax.numpy as jnp
from jax import lax
from jax.experimental import pallas as pl
from jax.experimental.pallas import tpu as pltpu
from jax.experimental.pallas import tpu_sc as plsc

C = 384
HW = 224 * 224
NC, NS, L = 2, 16, 16
NW = NC * NS
CPW = C // NW
NV = HW // L
STRIDE = 257
SUB = 1
NREG = L * SUB
HSCR = NREG * STRIDE + L

_GATHER_DNUMS = lax.GatherDimensionNumbers(
    offset_dims=(), collapsed_slice_dims=(0,), start_index_map=(0,))


def _lane_shuffle(v, idx):
    return lax.gather(v, idx[:, None], _GATHER_DNUMS, slice_sizes=(1,),
                      mode=lax.GatherScatterMode.PROMISE_IN_BOUNDS)


def _hist_body(inp, min_o, max_o, hist_o,
               buf0, buf1, histscr, rowbuf, minbuf, maxbuf,
               sem0, sem1):
    wid = lax.axis_index("s") * NC + lax.axis_index("c")
    base_ch = wid * CPW
    lane = lax.iota(jnp.int32, L)
    ones = jnp.ones((L,), jnp.float32)
    zeros = jnp.zeros((L,), jnp.float32)
    region_f = [((lane + s * L) * STRIDE).astype(jnp.float32)
                for s in range(SUB)]

    izeros = jnp.zeros((L,), jnp.int32)

    def zbody(i):
        histscr[pl.ds(i * L, L)] = izeros
    plsc.parallel_loop(0, HSCR // L, unroll=4)(zbody)

    bufs = (buf0, buf1)
    sems = (sem0, sem1)
    cur_cp = pltpu.async_copy(inp.at[base_ch], buf0, sem0)

    for j in range(CPW):
        cur = bufs[j % 2]
        nxt_cp = None
        if j + 1 < CPW:
            nxt_cp = pltpu.async_copy(inp.at[base_ch + j + 1],
                                      bufs[(j + 1) % 2], sems[(j + 1) % 2])
        cur_cp.wait()

        v0 = cur[pl.ds(0, L)]

        def abody(i, carry):
            mn0, mn1, mn2, mn3, mx0, mx1, mx2, mx3 = carry
            a = cur[pl.ds(i * L, L)]
            b = cur[pl.ds((i + 1) * L, L)]
            c = cur[pl.ds((i + 2) * L, L)]
            d = cur[pl.ds((i + 3) * L, L)]
            return (jnp.minimum(mn0, a), jnp.minimum(mn1, b),
                    jnp.minimum(mn2, c), jnp.minimum(mn3, d),
                    jnp.maximum(mx0, a), jnp.maximum(mx1, b),
                    jnp.maximum(mx2, c), jnp.maximum(mx3, d))

        mn0, mn1, mn2, mn3, mx0, mx1, mx2, mx3 = plsc.parallel_loop(
            0, NV, step=4, unroll=2,
            carry=(v0, v0, v0, v0, v0, v0, v0, v0))(abody)
        mn = jnp.minimum(jnp.minimum(mn0, mn1), jnp.minimum(mn2, mn3))
        mx = jnp.maximum(jnp.maximum(mx0, mx1), jnp.maximum(mx2, mx3))
        for k in (1, 2, 4, 8):
            perm = lane ^ k
            mn = jnp.minimum(mn, _lane_shuffle(mn, perm))
            mx = jnp.maximum(mx, _lane_shuffle(mx, perm))
        rng = mx - mn
        denom = jnp.where(rng > 0.0, rng, jnp.full((L,), 1.0, jnp.float32))
        scale = jnp.full((L,), 255.99975585937500, jnp.float32) / denom
        mnsc = mn * scale
        biases = [rf - mnsc for rf in region_f]

        lm = lane == j
        minbuf[...] = jnp.where(lm, mn, minbuf[...])
        maxbuf[...] = jnp.where(lm, mx, maxbuf[...])

        iones = jnp.ones((L,), jnp.int32)

        def bbody(i):
            v = cur[pl.ds(i * L, L)]
            t = v * scale + biases[0]
            plsc.addupdate_scatter(histscr, [t.astype(jnp.int32)], iones)
        plsc.parallel_loop(0, NV, unroll=16)(bbody)

        def cbody(g):
            acc = jnp.zeros((L,), jnp.int32)
            for q in range(NREG):
                acc = acc + histscr[pl.ds(q * STRIDE + g * L, L)]
                histscr[pl.ds(q * STRIDE + g * L, L)] = izeros
            rowbuf[pl.ds(g * L, L)] = acc.astype(jnp.float32)
        plsc.parallel_loop(0, 256 // L, unroll=2)(cbody)

        pltpu.sync_copy(rowbuf, hist_o.at[base_ch + j])
        cur_cp = nxt_cp

    pltpu.sync_copy(minbuf, min_o.at[wid])
    pltpu.sync_copy(maxbuf, max_o.at[wid])


_mesh = plsc.VectorSubcoreMesh(core_axis_name="c", subcore_axis_name="s",
                               num_cores=NC, num_subcores=NS)

_sc_hist = pl.kernel(
    _hist_body,
    out_type=(jax.ShapeDtypeStruct((NW, L), jnp.float32),
              jax.ShapeDtypeStruct((NW, L), jnp.float32),
              jax.ShapeDtypeStruct((C, 256), jnp.float32)),
    mesh=_mesh,
    scratch_types=(pltpu.VMEM((HW,), jnp.float32),
                   pltpu.VMEM((HW,), jnp.float32),
                   pltpu.VMEM((HSCR,), jnp.int32),
                   pltpu.VMEM((256,), jnp.float32),
                   pltpu.VMEM((L,), jnp.float32),
                   pltpu.VMEM((L,), jnp.float32),
                   pltpu.SemaphoreType.DMA,
                   pltpu.SemaphoreType.DMA),
    compiler_params=pltpu.CompilerParams(needs_layout_passes=False),
)


def kernel(input):
    x2 = input.reshape(C, HW)
    min_st, max_st, hist = _sc_hist(x2)
    target_min = min_st[:, :CPW].reshape(C)
    target_max = max_st[:, :CPW].reshape(C)
    return (input, hist, target_min, target_max)
